# baseline (speedup 1.0000x reference)
import functools

import numpy as np
import jax
import jax.numpy as jnp
from jax.experimental import pallas as pl
from jax.experimental.pallas import tpu as pltpu


def _tap_mask_matrix(H, W):
    m = np.zeros((H * W, 9), dtype=np.float32)
    for dy in range(3):
        y0, y1 = max(0, dy - 1), min(H, H - 1 + dy)
        for dx in range(3):
            x0, x1 = max(0, dx - 1), min(W, W - 1 + dx)
            blk = np.zeros((H, W), dtype=np.float32)
            blk[y0:y1, x0:x1] = 1.0
            m[:, dy * 3 + dx] = blk.reshape(-1)
    return jnp.asarray(m)


def _fused_kernel(f1_ref, f2_ref, f3_ref, m_ref, w_ref, bias_ref, o_ref,
                  *, inv_hw, C):
    f1 = f1_ref[0]
    f2 = f2_ref[0]
    f = f1 + f2 + f3_ref[0]
    s = jnp.dot(f, m_ref[...], preferred_element_type=jnp.float32)
    lane = jax.lax.broadcasted_iota(jnp.int32, s.shape, 1)
    acc = jnp.zeros((128, 1), jnp.float32)
    for t in range(9):
        col = jnp.sum(jnp.where(lane == t, s, 0.0), axis=1,
                      keepdims=True)
        acc = acc + jnp.dot(w_ref[t * 128:(t + 1) * 128, 0:C],
                            col, preferred_element_type=jnp.float32)
    g = acc * inv_hw + bias_ref[0:128]
    comp = jnp.dot(w_ref[1152:1216, :], g,
                   preferred_element_type=jnp.float32) + bias_ref[128:192]
    e = jnp.dot(w_ref[1216:1600, 0:64], comp,
                preferred_element_type=jnp.float32) + bias_ref[192:192 + 3 * C]
    l0 = e[0:C]
    l1 = e[C:2 * C]
    l2 = e[2 * C:3 * C]
    m = jnp.maximum(jnp.maximum(l0, l1), l2)
    e0 = jnp.exp(l0 - m)
    e1 = jnp.exp(l1 - m)
    e2 = jnp.exp(l2 - m)
    inv = 1.0 / (e0 + e1 + e2)
    a = e0 * inv
    bc = (e1 + e2) * inv
    o_ref[0] = f1 * a + f2 * bc


def kernel(feature1, feature2, feature3,
           w_conv, b_conv, w_comp, b_comp, w_exp, b_exp):
    B, C, H, W = feature1.shape
    HW = H * W

    f1r = feature1.reshape(B, C, HW).astype(jnp.float32)
    f2r = feature2.reshape(B, C, HW).astype(jnp.float32)
    f3r = feature3.reshape(B, C, HW).astype(jnp.float32)
    mask = _tap_mask_matrix(H, W)

    w_convT = jnp.transpose(w_conv.reshape(C, 9, 128), (1, 2, 0))
    def _pad128(x):
        return jnp.pad(x, ((0, 0), (0, 128 - x.shape[1])))
    w_packed = jnp.concatenate([
        _pad128(w_convT.reshape(9 * 128, C)),
        _pad128(jnp.transpose(w_comp)),
        _pad128(jnp.transpose(w_exp)),
    ], axis=0)
    bias_packed = jnp.concatenate([
        jnp.transpose(b_conv),
        jnp.transpose(b_comp),
        jnp.transpose(b_exp),
    ], axis=0)

    out = pl.pallas_call(
        functools.partial(_fused_kernel, inv_hw=1.0 / float(HW), C=C),
        out_shape=jax.ShapeDtypeStruct((B, C, HW), jnp.float32),
        grid=(B,),
        in_specs=[
            pl.BlockSpec((1, C, HW), lambda b: (b, 0, 0)),
            pl.BlockSpec((1, C, HW), lambda b: (b, 0, 0)),
            pl.BlockSpec((1, C, HW), lambda b: (b, 0, 0)),
            pl.BlockSpec((HW, 9), lambda b: (0, 0)),
            pl.BlockSpec((1600, 128), lambda b: (0, 0)),
            pl.BlockSpec((192 + 3 * C, 1), lambda b: (0, 0)),
        ],
        out_specs=pl.BlockSpec((1, C, HW), lambda b: (b, 0, 0)),
        compiler_params=pltpu.CompilerParams(
            dimension_semantics=("parallel",)),
    )(f1r, f2r, f3r, mask, w_packed, bias_packed)

    return out.reshape(B, C, H, W)


# speedup vs baseline: 1.0564x; 1.0564x over previous
import functools

import numpy as np
import jax
import jax.numpy as jnp
from jax.experimental import pallas as pl
from jax.experimental.pallas import tpu as pltpu


def _tap_mask_matrix(H, W):
    m = np.zeros((H * W, 9), dtype=np.float32)
    for dy in range(3):
        y0, y1 = max(0, dy - 1), min(H, H - 1 + dy)
        for dx in range(3):
            x0, x1 = max(0, dx - 1), min(W, W - 1 + dx)
            blk = np.zeros((H, W), dtype=np.float32)
            blk[y0:y1, x0:x1] = 1.0
            m[:, dy * 3 + dx] = blk.reshape(-1)
    return jnp.asarray(m)


def _fused_kernel(f1_ref, f2_ref, f3_ref, m_ref, w_ref, bias_ref, o_ref,
                  *, inv_hw, C):
    f1 = f1_ref[0]
    f2 = f2_ref[0]
    f = f1 + f2 + f3_ref[0]
    s = jnp.dot(f, m_ref[...], preferred_element_type=jnp.float32)
    lane = jax.lax.broadcasted_iota(jnp.int32, s.shape, 1)
    acc = jnp.zeros((128, 1), jnp.float32)
    for t in range(9):
        col = jnp.sum(jnp.where(lane == t, s, 0.0), axis=1,
                      keepdims=True)
        acc = acc + jnp.dot(w_ref[t * 128:(t + 1) * 128, 0:C],
                            col, preferred_element_type=jnp.float32)
    g = acc * inv_hw + bias_ref[0:128]
    comp = jnp.dot(w_ref[1152:1216, :], g,
                   preferred_element_type=jnp.float32) + bias_ref[128:192]
    e = jnp.dot(w_ref[1216:1600, 0:64], comp,
                preferred_element_type=jnp.float32) + bias_ref[192:192 + 3 * C]
    l0 = e[0:C]
    l1 = e[C:2 * C]
    l2 = e[2 * C:3 * C]
    m = jnp.maximum(jnp.maximum(l0, l1), l2)
    e0 = jnp.exp(l0 - m)
    e1 = jnp.exp(l1 - m)
    e2 = jnp.exp(l2 - m)
    inv = 1.0 / (e0 + e1 + e2)
    a = e0 * inv
    bc = (e1 + e2) * inv
    o_ref[0] = (f1 * a + f2 * bc).astype(jnp.bfloat16)


def kernel(feature1, feature2, feature3,
           w_conv, b_conv, w_comp, b_comp, w_exp, b_exp):
    B, C, H, W = feature1.shape
    HW = H * W

    f1r = feature1.reshape(B, C, HW).astype(jnp.float32)
    f2r = feature2.reshape(B, C, HW).astype(jnp.float32)
    f3r = feature3.reshape(B, C, HW).astype(jnp.float32)
    mask = _tap_mask_matrix(H, W)

    w_convT = jnp.transpose(w_conv.reshape(C, 9, 128), (1, 2, 0))
    def _pad128(x):
        return jnp.pad(x, ((0, 0), (0, 128 - x.shape[1])))
    w_packed = jnp.concatenate([
        _pad128(w_convT.reshape(9 * 128, C)),
        _pad128(jnp.transpose(w_comp)),
        _pad128(jnp.transpose(w_exp)),
    ], axis=0)
    bias_packed = jnp.concatenate([
        jnp.transpose(b_conv),
        jnp.transpose(b_comp),
        jnp.transpose(b_exp),
    ], axis=0)

    out = pl.pallas_call(
        functools.partial(_fused_kernel, inv_hw=1.0 / float(HW), C=C),
        out_shape=jax.ShapeDtypeStruct((B, C, HW), jnp.bfloat16),
        grid=(B,),
        in_specs=[
            pl.BlockSpec((1, C, HW), lambda b: (b, 0, 0)),
            pl.BlockSpec((1, C, HW), lambda b: (b, 0, 0)),
            pl.BlockSpec((1, C, HW), lambda b: (b, 0, 0)),
            pl.BlockSpec((HW, 9), lambda b: (0, 0)),
            pl.BlockSpec((1600, 128), lambda b: (0, 0)),
            pl.BlockSpec((192 + 3 * C, 1), lambda b: (0, 0)),
        ],
        out_specs=pl.BlockSpec((1, C, HW), lambda b: (b, 0, 0)),
        compiler_params=pltpu.CompilerParams(
            dimension_semantics=("parallel",)),
    )(f1r, f2r, f3r, mask, w_packed, bias_packed)

    return out.astype(jnp.float32).reshape(B, C, H, W)


# speedup vs baseline: 1.0649x; 1.0080x over previous
import functools

import numpy as np
import jax
import jax.numpy as jnp
from jax.experimental import pallas as pl
from jax.experimental.pallas import tpu as pltpu


def _tap_mask_matrix(H, W):
    m = np.zeros((H * W, 9), dtype=np.float32)
    for dy in range(3):
        y0, y1 = max(0, dy - 1), min(H, H - 1 + dy)
        for dx in range(3):
            x0, x1 = max(0, dx - 1), min(W, W - 1 + dx)
            blk = np.zeros((H, W), dtype=np.float32)
            blk[y0:y1, x0:x1] = 1.0
            m[:, dy * 3 + dx] = blk.reshape(-1)
    return jnp.asarray(m)


def _fused_kernel(f1_ref, f2_ref, f3_ref, m_ref, w_ref, bias_ref, o_ref,
                  *, inv_hw, C, NB):
    for i in range(NB):
        f1 = f1_ref[i]
        f2 = f2_ref[i]
        f = f1 + f2 + f3_ref[i]
        s = jnp.dot(f, m_ref[...],
                    preferred_element_type=jnp.float32)
        lane = jax.lax.broadcasted_iota(jnp.int32, s.shape, 1)
        acc = jnp.zeros((128, 1), jnp.float32)
        for t in range(9):
            col = jnp.sum(jnp.where(lane == t, s, 0.0), axis=1,
                          keepdims=True)
            acc = acc + jnp.dot(w_ref[t * 128:(t + 1) * 128, 0:C],
                                col, preferred_element_type=jnp.float32)
        g = acc * inv_hw + bias_ref[0:128]
        comp = jnp.dot(w_ref[1152:1216, :], g,
                       preferred_element_type=jnp.float32) + bias_ref[128:192]
        e = jnp.dot(w_ref[1216:1600, 0:64], comp,
                    preferred_element_type=jnp.float32) + bias_ref[192:192 + 3 * C]
        l0 = e[0:C]
        l1 = e[C:2 * C]
        l2 = e[2 * C:3 * C]
        m = jnp.maximum(jnp.maximum(l0, l1), l2)
        e0 = jnp.exp(l0 - m)
        e1 = jnp.exp(l1 - m)
        e2 = jnp.exp(l2 - m)
        inv = 1.0 / (e0 + e1 + e2)
        a = e0 * inv
        bc = (e1 + e2) * inv
        o_ref[i] = (f1 * a + f2 * bc).astype(jnp.bfloat16)


def kernel(feature1, feature2, feature3,
           w_conv, b_conv, w_comp, b_comp, w_exp, b_exp):
    B, C, H, W = feature1.shape
    HW = H * W

    f1r = feature1.reshape(B, C, HW).astype(jnp.float32)
    f2r = feature2.reshape(B, C, HW).astype(jnp.float32)
    f3r = feature3.reshape(B, C, HW).astype(jnp.float32)
    mask = _tap_mask_matrix(H, W)

    w_convT = jnp.transpose(w_conv.reshape(C, 9, 128), (1, 2, 0))
    def _pad128(x):
        return jnp.pad(x, ((0, 0), (0, 128 - x.shape[1])))
    w_packed = jnp.concatenate([
        _pad128(w_convT.reshape(9 * 128, C)),
        _pad128(jnp.transpose(w_comp)),
        _pad128(jnp.transpose(w_exp)),
    ], axis=0)
    bias_packed = jnp.concatenate([
        jnp.transpose(b_conv),
        jnp.transpose(b_comp),
        jnp.transpose(b_exp),
    ], axis=0)

    NB = 2 if B % 2 == 0 else 1
    out = pl.pallas_call(
        functools.partial(_fused_kernel, inv_hw=1.0 / float(HW), C=C, NB=NB),
        out_shape=jax.ShapeDtypeStruct((B, C, HW), jnp.bfloat16),
        grid=(B // NB,),
        in_specs=[
            pl.BlockSpec((NB, C, HW), lambda b: (b, 0, 0)),
            pl.BlockSpec((NB, C, HW), lambda b: (b, 0, 0)),
            pl.BlockSpec((NB, C, HW), lambda b: (b, 0, 0)),
            pl.BlockSpec((HW, 9), lambda b: (0, 0)),
            pl.BlockSpec((1600, 128), lambda b: (0, 0)),
            pl.BlockSpec((192 + 3 * C, 1), lambda b: (0, 0)),
        ],
        out_specs=pl.BlockSpec((NB, C, HW), lambda b: (b, 0, 0)),
        compiler_params=pltpu.CompilerParams(
            dimension_semantics=("parallel",)),
    )(f1r, f2r, f3r, mask, w_packed, bias_packed)

    return out.astype(jnp.float32).reshape(B, C, H, W)
